# baseline (speedup 1.0000x reference)
import functools

import jax
import jax.numpy as jnp
from jax import lax
from jax.experimental import pallas as pl
from jax.experimental.pallas import tpu as pltpu
from jax.experimental.pallas import tpu_sc as plsc

F32 = jnp.float32

_B = 16384
_F = 26
_D = 16
_NC = 2
_NS = 16
_NW = _NC * _NS
_PW = _B // _NW
_C = 128
_NK = _PW // _C
_G = _C // 16


@functools.partial(
    pl.kernel,
    out_type=jax.ShapeDtypeStruct((_B,), F32),
    mesh=plsc.VectorSubcoreMesh(core_axis_name="c", subcore_axis_name="s"),
    compiler_params=pltpu.CompilerParams(
        needs_layout_passes=False, use_tc_tiling_on_sc=False
    ),
    scratch_types=[
        pltpu.VMEM((_C, _F), jnp.int32),
        pltpu.VMEM((_F, _C), jnp.int32),
        pltpu.VMEM((_F, _C, _D), F32),
        pltpu.VMEM((_F, _C), F32),
        pltpu.VMEM((16,), F32),
        pltpu.VMEM((_PW,), F32),
        pltpu.SemaphoreType.DMA,
    ],
)
def _fm_sc(x, emb, lin, bias16, out, xblk_v, idx_v, rows_v, lin_v, bias_v,
           out_v, sem):
    wid = lax.axis_index("s") * _NC + lax.axis_index("c")
    base = wid * _PW

    pltpu.sync_copy(bias16, bias_v)
    bias_vec = bias_v[...]
    iota16 = lax.iota(jnp.int32, 16)

    for k in range(_NK):
        r0 = base + k * _C
        pltpu.sync_copy(x.at[pl.ds(r0, _C), :], xblk_v)
        for f in range(_F):
            f_splat = jnp.full((16,), f, jnp.int32)
            for g in range(_G):
                rows16 = g * 16 + iota16
                v = plsc.load_gather(xblk_v, [rows16, f_splat])
                idx_v[f, pl.ds(g * 16, 16)] = v
        cps = []
        for f in range(_F):
            cps.append(pltpu.async_copy(emb.at[idx_v.at[f]], rows_v.at[f], sem))
            cps.append(pltpu.async_copy(lin.at[idx_v.at[f]], lin_v.at[f], sem))
        for cp in cps:
            cp.wait()

        def group_body(g, carry):
            row0 = g * 16
            rows16 = row0 + iota16
            lin_acc = jnp.zeros((16,), F32)
            for f in range(_F):
                lin_acc = lin_acc + lin_v[f, pl.ds(row0, 16)]
            inter = jnp.zeros((16,), F32)
            for d in range(_D):
                sd = jnp.zeros((16,), F32)
                ssd = jnp.zeros((16,), F32)
                d_splat = jnp.full((16,), d, jnp.int32)
                for f in range(_F):
                    f_splat = jnp.full((16,), f, jnp.int32)
                    e = plsc.load_gather(rows_v, [f_splat, rows16, d_splat])
                    sd = sd + e
                    ssd = ssd + e * e
                inter = inter + (sd * sd - ssd)
            z = lin_acc + bias_vec + 0.5 * inter
            out_v[pl.ds(k * _C + row0, 16)] = 1.0 / (1.0 + jnp.exp(-z))
            return carry

        lax.fori_loop(0, _G, group_body, 0)

    pltpu.sync_copy(out_v, out.at[pl.ds(base, _PW)])


def kernel(x, emb_table, linear_table, bias):
    lin_flat = linear_table.reshape(-1).astype(F32)
    bias16 = jnp.broadcast_to(bias.astype(F32), (16,))
    out = _fm_sc(x.astype(jnp.int32), emb_table.astype(F32), lin_flat, bias16)
    return out.reshape(_B, 1)


# speedup vs baseline: 2.0024x; 2.0024x over previous
import functools

import jax
import jax.numpy as jnp
from jax import lax
from jax.experimental import pallas as pl
from jax.experimental.pallas import tpu as pltpu
from jax.experimental.pallas import tpu_sc as plsc

F32 = jnp.float32

_B = 16384
_F = 26
_D = 16
_V = 1000000
_NC = 2
_NS = 16
_NW = _NC * _NS
_PW = _B // _NW
_C = 128
_NK = _PW // _C
_G = _C // 16

_PANEL = 1536
_NP = 999936 // _PANEL
_PPW = (_NP + _NW - 1) // _NW
_TAIL0 = _NP * _PANEL
_TAIL = _V - _TAIL0

_mesh = plsc.VectorSubcoreMesh(core_axis_name="c", subcore_axis_name="s")


@functools.partial(
    pl.kernel,
    out_type=jax.ShapeDtypeStruct((_V * _D,), F32),
    mesh=_mesh,
    compiler_params=pltpu.CompilerParams(
        needs_layout_passes=False, use_tc_tiling_on_sc=True
    ),
    scratch_types=[
        pltpu.VMEM((_D, _PANEL), F32),
        pltpu.VMEM((_D, _PANEL), F32),
        pltpu.VMEM((_PANEL * _D,), F32),
        pltpu.SemaphoreType.DMA,
        pltpu.SemaphoreType.DMA,
    ],
)
def _detile_sc(embT, tail_flat, out, pan_a, pan_b, rowbuf, sem_a, sem_b):
    wid = lax.axis_index("s") * _NC + lax.axis_index("c")
    iota16 = lax.iota(jnp.int32, 16)
    scat16 = iota16 * _D

    def panel_id(slot):
        return wid + _NW * slot

    def start_load(slot, pan, sem):
        p = panel_id(slot)

        @pl.when(p < _NP)
        def _():
            pltpu.async_copy(embT.at[:, pl.ds(p * _PANEL, _PANEL)], pan, sem)

    def do_panel(slot, pan, sem):
        p = panel_id(slot)

        @pl.when(p < _NP)
        def _():
            pltpu.make_async_copy(
                embT.at[:, pl.ds(p * _PANEL, _PANEL)], pan, sem
            ).wait()

            def col_group(j, carry):
                base = j * (16 * _D)
                for d in range(_D):
                    v = pan[d, pl.ds(j * 16, 16)]
                    plsc.store_scatter(rowbuf, [base + scat16 + d], v)
                return carry

            lax.fori_loop(0, _PANEL // 16, col_group, 0)
            pltpu.sync_copy(rowbuf, out.at[pl.ds(p * _PANEL * _D, _PANEL * _D)])

    for half in range(_PPW // 2 + 1):
        s0, s1 = 2 * half, 2 * half + 1
        if s0 < _PPW:
            start_load(s0, pan_a, sem_a)
        if s1 < _PPW:
            start_load(s1, pan_b, sem_b)
        if s0 < _PPW:
            do_panel(s0, pan_a, sem_a)
        if s1 < _PPW:
            do_panel(s1, pan_b, sem_b)

    @pl.when(wid == _NW - 1)
    def _():
        pltpu.sync_copy(tail_flat, rowbuf.at[pl.ds(0, _TAIL * _D)])
        pltpu.sync_copy(
            rowbuf.at[pl.ds(0, _TAIL * _D)],
            out.at[pl.ds(_TAIL0 * _D, _TAIL * _D)],
        )


@functools.partial(
    pl.kernel,
    out_type=jax.ShapeDtypeStruct((_B,), F32),
    mesh=_mesh,
    compiler_params=pltpu.CompilerParams(
        needs_layout_passes=False, use_tc_tiling_on_sc=False
    ),
    scratch_types=[
        pltpu.VMEM((_F, _C), jnp.int32),
        pltpu.VMEM((_F, _C, _D), F32),
        pltpu.VMEM((_F, _C), F32),
        pltpu.VMEM((16,), F32),
        pltpu.VMEM((_PW,), F32),
        pltpu.SemaphoreType.DMA,
    ],
)
def _fm_sc(xg, emb, lin, bias16, out, idx_v, rows_v, lin_v, bias_v, out_v, sem):
    wid = lax.axis_index("s") * _NC + lax.axis_index("c")
    base = wid * _PW

    pltpu.sync_copy(bias16, bias_v)
    bias_vec = bias_v[...]
    iota16 = lax.iota(jnp.int32, 16)

    for k in range(_NK):
        pltpu.sync_copy(xg.at[wid, k], idx_v)
        cps = []
        for f in range(_F):
            cps.append(pltpu.async_copy(emb.at[idx_v.at[f]], rows_v.at[f], sem))
            cps.append(pltpu.async_copy(lin.at[idx_v.at[f]], lin_v.at[f], sem))
        for cp in cps:
            cp.wait()

        def group_body(g, carry):
            row0 = g * 16
            rows16 = row0 + iota16
            lin_acc = jnp.zeros((16,), F32)
            for f in range(_F):
                lin_acc = lin_acc + lin_v[f, pl.ds(row0, 16)]
            inter = jnp.zeros((16,), F32)
            for d in range(_D):
                sd = jnp.zeros((16,), F32)
                ssd = jnp.zeros((16,), F32)
                d_splat = jnp.full((16,), d, jnp.int32)
                for f in range(_F):
                    f_splat = jnp.full((16,), f, jnp.int32)
                    e = plsc.load_gather(rows_v, [f_splat, rows16, d_splat])
                    sd = sd + e
                    ssd = ssd + e * e
                inter = inter + (sd * sd - ssd)
            z = lin_acc + bias_vec + 0.5 * inter
            out_v[pl.ds(k * _C + row0, 16)] = 1.0 / (1.0 + jnp.exp(-z))
            return carry

        lax.fori_loop(0, _G, group_body, 0)

    pltpu.sync_copy(out_v, out.at[pl.ds(base, _PW)])


def kernel(x, emb_table, linear_table, bias):
    emb_f = emb_table.astype(F32)
    tail_flat = emb_f[_TAIL0:, :].reshape(-1)
    emb_lin = _detile_sc(emb_f.T, tail_flat)
    emb2d = emb_lin.reshape(_V, _D)
    xg = x.astype(jnp.int32).reshape(_NW, _NK, _C, _F).transpose(0, 1, 3, 2)
    lin_flat = linear_table.reshape(-1).astype(F32)
    bias16 = jnp.broadcast_to(bias.astype(F32), (16,))
    out = _fm_sc(xg, emb2d, lin_flat, bias16)
    return out.reshape(_B, 1)
